# baseline (speedup 1.0000x reference)
import functools

import jax
import jax.numpy as jnp
from jax import lax
from jax.experimental import pallas as pl
from jax.experimental.pallas import tpu as pltpu, tpu_sc as plsc

_G = 16
_S = 16384
_SHIFT = 14


@functools.lru_cache(maxsize=None)
def _make_kernels(V, D, B):
    info = plsc.get_sparse_core_info()
    NC, NS = info.num_cores, info.num_subcores
    NW = NC * NS
    BPW = B // NW
    NCH = BPW // _G
    CH = BPW // 128
    R = 34 * _S
    NBLK = -(-(V - R) // _S)
    V2P = NBLK * _S
    assert B % (NW * _G) == 0 and NCH % 2 == 0 and D == 64
    assert V % 8 == 0 and R % 8 == 0 and R % _S == 0 and (1 << _SHIFT) == _S
    mesh = plsc.VectorSubcoreMesh(core_axis_name="c", subcore_axis_name="s")

    @functools.partial(
        pl.kernel,
        mesh=mesh,
        out_type=jax.ShapeDtypeStruct((B, D), jnp.float32),
        scratch_types=[
            pltpu.VMEM((BPW,), jnp.int32),
            pltpu.VMEM((2, _G, 8, D), jnp.float32),
            pltpu.VMEM((BPW, D), jnp.float32),
            pltpu.SemaphoreType.DMA,
            pltpu.SemaphoreType.DMA,
        ],
    )
    def ka(idx_hbm, tbl_hbm, out_hbm, idx_v, tiles_v, sel_v, sem0, sem1):
        wid = lax.axis_index("s") * NC + lax.axis_index("c")
        base = wid * BPW
        pltpu.sync_copy(idx_hbm.at[pl.ds(base, BPW)], idx_v)
        sems = [sem0, sem1]

        def fire(ch, buf):
            vvec = idx_v[pl.ds(pl.multiple_of(ch * _G, _G), _G)]
            for j in range(_G):
                v8 = jnp.minimum(lax.shift_right_logical(vvec[j], 3), R // 8 - 1)
                pltpu.async_copy(tbl_hbm.at[v8], tiles_v.at[buf, j], sems[buf])

        def drain(buf):
            for j in range(_G):
                pltpu.make_async_copy(
                    tbl_hbm.at[0], tiles_v.at[buf, j], sems[buf]
                ).wait()

        def select(ch, buf):
            o = pl.multiple_of(ch * _G, _G)
            vvec = idx_v[pl.ds(o, _G)]
            for j in range(_G):
                par = lax.bitwise_and(vvec[j], 7)
                for g in range(D // 16):
                    sel_v[o + j, pl.ds(g * 16, 16)] = (
                        tiles_v[buf, j, par, pl.ds(g * 16, 16)]
                    )

        fire(0, 0)

        @pl.loop(0, NCH // 2)
        def _(t):
            ch0 = t * 2
            fire(ch0 + 1, 1)
            drain(0)
            select(ch0, 0)

            @pl.when(ch0 + 2 < NCH)
            def _():
                fire(ch0 + 2, 0)

            drain(1)
            select(ch0 + 1, 1)

        pltpu.sync_copy(sel_v, out_hbm.at[pl.ds(base, BPW)])

    @functools.partial(
        pl.kernel,
        mesh=mesh,
        compiler_params=pltpu.CompilerParams(use_tc_tiling_on_sc=False),
        out_type=jax.ShapeDtypeStruct((D, B), jnp.float32),
        scratch_types=[
            pltpu.VMEM((BPW,), jnp.int32),
            pltpu.VMEM((D, CH, 128), jnp.int32),
            pltpu.VMEM((D, BPW), jnp.float32),
            pltpu.SemaphoreType.DMA,
        ],
    )
    def kb(idx_hbm, tbl_hbm, out_hbm, idx_v, lists_v, rows_v, sem):
        wid = lax.axis_index("s") * NC + lax.axis_index("c")
        base = wid * BPW
        pltpu.sync_copy(idx_hbm.at[pl.ds(base, BPW)], idx_v)
        for g in range(BPW // 16):
            v = idx_v[pl.ds(g * 16, 16)]
            vv = jnp.clip(v - R, 0, V - R - 1)
            c = lax.shift_right_logical(vv, _SHIFT)
            w = lax.bitwise_and(vv, _S - 1)
            cw = c * (D * _S) + w

            @pl.loop(0, D, unroll=8)
            def _(e):
                lists_v[e, g // 8, pl.ds((g % 8) * 16, 16)] = cw + e * _S

        @pl.loop(0, D, unroll=4)
        def _(e):
            copies = [
                pltpu.async_copy(
                    tbl_hbm.at[lists_v.at[e, j]],
                    rows_v.at[e, pl.ds(j * 128, 128)],
                    sem,
                )
                for j in range(CH)
            ]
            for c2 in copies:
                c2.wait()

        pltpu.sync_copy(rows_v, out_hbm.at[:, pl.ds(base, BPW)])

    @functools.partial(
        pl.pallas_call,
        grid=(NBLK, D),
        in_specs=[pl.BlockSpec((8, _S), lambda c, e: (e // 8, R // _S + c))],
        out_specs=pl.BlockSpec((_S,), lambda c, e: (c * D + e,)),
        out_shape=jax.ShapeDtypeStruct((D * V2P,), jnp.float32),
    )
    def depad(in_ref, out_ref):
        e = pl.program_id(1)
        out_ref[...] = in_ref[lax.rem(e, 8), :]

    def run(indices, w):
        idx = indices.astype(jnp.int32)
        p1 = w[:R].reshape(R // 8, 8, D)
        p2 = depad(w.T)
        o1 = ka(idx, p1)
        o2t = kb(idx, p2)
        return jnp.where((idx < R)[:, None], o1, o2t.T)

    return run


def kernel(indices, embedding_weight):
    V, D = embedding_weight.shape
    (B,) = indices.shape
    return _make_kernels(V, D, B)(indices, embedding_weight)


# speedup vs baseline: 1.5379x; 1.5379x over previous
import functools

import jax
import jax.numpy as jnp
from jax import lax
from jax.experimental import pallas as pl
from jax.experimental.pallas import tpu as pltpu, tpu_sc as plsc

_G = 16
_S = 16384
_SHIFT = 14


@functools.lru_cache(maxsize=None)
def _make_kernels(V, D, B):
    info = plsc.get_sparse_core_info()
    NC, NS = info.num_cores, info.num_subcores
    NW = NC * NS
    BPW = B // NW
    NCH = BPW // _G
    CH = BPW // 128
    R = 34 * _S
    NBLK = -(-(V - R) // _S)
    V2P = NBLK * _S
    assert B % (NW * _G) == 0 and NCH % 2 == 0 and D == 64
    assert V % 8 == 0 and R % 8 == 0 and R % _S == 0 and (1 << _SHIFT) == _S
    mesh = plsc.VectorSubcoreMesh(core_axis_name="c", subcore_axis_name="s")

    @functools.partial(
        pl.kernel,
        mesh=mesh,
        out_type=jax.ShapeDtypeStruct((B, D), jnp.float32),
        scratch_types=[
            pltpu.VMEM((BPW,), jnp.int32),
            pltpu.VMEM((2, _G, 8, D), jnp.float32),
            pltpu.VMEM((BPW, D), jnp.float32),
            pltpu.SemaphoreType.DMA,
            pltpu.SemaphoreType.DMA,
        ],
    )
    def ka(idx_hbm, tbl_hbm, out_hbm, idx_v, tiles_v, sel_v, sem0, sem1):
        wid = lax.axis_index("s") * NC + lax.axis_index("c")
        base = wid * BPW
        pltpu.sync_copy(idx_hbm.at[pl.ds(base, BPW)], idx_v)
        sems = [sem0, sem1]

        def fire(ch, buf):
            vvec = idx_v[pl.ds(pl.multiple_of(ch * _G, _G), _G)]
            for j in range(_G):
                v8 = jnp.minimum(lax.shift_right_logical(vvec[j], 3), R // 8 - 1)
                pltpu.async_copy(tbl_hbm.at[v8], tiles_v.at[buf, j], sems[buf])

        def drain(buf):
            for j in range(_G):
                pltpu.make_async_copy(
                    tbl_hbm.at[0], tiles_v.at[buf, j], sems[buf]
                ).wait()

        def select(ch, buf):
            o = pl.multiple_of(ch * _G, _G)
            vvec = idx_v[pl.ds(o, _G)]
            for j in range(_G):
                par = lax.bitwise_and(vvec[j], 7)
                for g in range(D // 16):
                    sel_v[o + j, pl.ds(g * 16, 16)] = (
                        tiles_v[buf, j, par, pl.ds(g * 16, 16)]
                    )

        fire(0, 0)

        @pl.loop(0, NCH // 2)
        def _(t):
            ch0 = t * 2
            fire(ch0 + 1, 1)
            drain(0)
            select(ch0, 0)

            @pl.when(ch0 + 2 < NCH)
            def _():
                fire(ch0 + 2, 0)

            drain(1)
            select(ch0 + 1, 1)

        pltpu.sync_copy(sel_v, out_hbm.at[pl.ds(base, BPW)])

    @functools.partial(
        pl.kernel,
        mesh=mesh,
        compiler_params=pltpu.CompilerParams(use_tc_tiling_on_sc=False),
        out_type=jax.ShapeDtypeStruct((D, B), jnp.float32),
        scratch_types=[
            pltpu.VMEM((BPW,), jnp.int32),
            pltpu.VMEM((8, CH, 128), jnp.int32),
            pltpu.VMEM((D, BPW), jnp.float32),
            pltpu.SemaphoreType.DMA,
        ],
    )
    def kb(idx_hbm, *args):
        tbls = args[:8]
        out_hbm, idx_v, lists_v, rows_v, sem = args[8:]
        wid = lax.axis_index("s") * NC + lax.axis_index("c")
        base = wid * BPW
        pltpu.sync_copy(idx_hbm.at[pl.ds(base, BPW)], idx_v)
        for g in range(BPW // 16):
            v = idx_v[pl.ds(g * 16, 16)]
            vv = jnp.clip(v - R, 0, V - R - 1)
            c = lax.shift_right_logical(vv, _SHIFT)
            w = lax.bitwise_and(vv, _S - 1)
            cw = c * (8 * _S) + w
            for k in range(8):
                lists_v[k, g // 8, pl.ds((g % 8) * 16, 16)] = cw + k * _S

        for r in range(8):
            tbl_r = tbls[r]

            @pl.loop(0, 8)
            def _(k):
                e = 8 * k + r
                copies = [
                    pltpu.async_copy(
                        tbl_r.at[lists_v.at[k, j]],
                        rows_v.at[e, pl.ds(j * 128, 128)],
                        sem,
                    )
                    for j in range(CH)
                ]
                for c2 in copies:
                    c2.wait()

        pltpu.sync_copy(rows_v, out_hbm.at[:, pl.ds(base, BPW)])

    @functools.partial(
        pl.pallas_call,
        grid=(NBLK, 8),
        in_specs=[pl.BlockSpec((8, _S), lambda c, k: (k, R // _S + c))],
        out_specs=tuple(
            pl.BlockSpec((_S,), lambda c, k: (c * 8 + k,)) for _ in range(8)
        ),
        out_shape=tuple(
            jax.ShapeDtypeStruct((8 * V2P,), jnp.float32) for _ in range(8)
        ),
    )
    def depad(in_ref, *out_refs):
        for r in range(8):
            out_refs[r][...] = in_ref[r, :]

    def run(indices, w):
        idx = indices.astype(jnp.int32)
        p1 = w[:R].reshape(R // 8, 8, D)
        p2s = depad(w.T)
        o1 = ka(idx, p1)
        o2t = kb(idx, *p2s)
        return jnp.where((idx < R)[:, None], o1, o2t.T)

    return run


def kernel(indices, embedding_weight):
    V, D = embedding_weight.shape
    (B,) = indices.shape
    return _make_kernels(V, D, B)(indices, embedding_weight)


# speedup vs baseline: 3.1584x; 2.0537x over previous
import functools

import jax
import jax.numpy as jnp
from jax import lax
from jax.experimental import pallas as pl
from jax.experimental.pallas import tpu as pltpu, tpu_sc as plsc

_G = 16
_S = 16384
_SHIFT = 14


@functools.lru_cache(maxsize=None)
def _make_kernels(V, D, B):
    info = plsc.get_sparse_core_info()
    NC, NS = info.num_cores, info.num_subcores
    NW = NC * NS
    BPW = B // NW
    NCH = BPW // _G
    CH = BPW // 128
    R = 34 * _S
    NBLK = -(-(V - R) // _S)
    V2P = NBLK * _S
    assert B % (NW * _G) == 0 and NCH % 2 == 0 and D == 64
    assert V % 8 == 0 and R % 8 == 0 and R % _S == 0 and (1 << _SHIFT) == _S
    mesh = plsc.VectorSubcoreMesh(core_axis_name="c", subcore_axis_name="s")

    @functools.partial(
        pl.kernel,
        mesh=mesh,
        out_type=jax.ShapeDtypeStruct((B, D), jnp.float32),
        scratch_types=[
            pltpu.VMEM((BPW,), jnp.int32),
            pltpu.VMEM((2, _G, 8, D), jnp.float32),
            pltpu.VMEM((BPW, D), jnp.float32),
            pltpu.SemaphoreType.DMA,
            pltpu.SemaphoreType.DMA,
        ],
    )
    def ka(idx_hbm, tbl_hbm, out_hbm, idx_v, tiles_v, sel_v, sem0, sem1):
        wid = lax.axis_index("s") * NC + lax.axis_index("c")
        base = wid * BPW
        pltpu.sync_copy(idx_hbm.at[pl.ds(base, BPW)], idx_v)
        sems = [sem0, sem1]

        def fire(ch, buf):
            vvec = idx_v[pl.ds(pl.multiple_of(ch * _G, _G), _G)]
            for j in range(_G):
                vj = vvec[j]
                vs = lax.shift_right_logical(vj, 3)
                v8 = jnp.where(vj < R, vs, lax.bitwise_and(vs, 0xFFFF))
                pltpu.async_copy(tbl_hbm.at[v8], tiles_v.at[buf, j], sems[buf])

        def drain(buf):
            for j in range(_G):
                pltpu.make_async_copy(
                    tbl_hbm.at[0], tiles_v.at[buf, j], sems[buf]
                ).wait()

        def select(ch, buf):
            o = pl.multiple_of(ch * _G, _G)
            vvec = idx_v[pl.ds(o, _G)]
            for j in range(_G):
                par = lax.bitwise_and(vvec[j], 7)
                for g in range(D // 16):
                    sel_v[o + j, pl.ds(g * 16, 16)] = (
                        tiles_v[buf, j, par, pl.ds(g * 16, 16)]
                    )

        fire(0, 0)

        @pl.loop(0, NCH // 2)
        def _(t):
            ch0 = t * 2
            fire(ch0 + 1, 1)
            drain(0)
            select(ch0, 0)

            @pl.when(ch0 + 2 < NCH)
            def _():
                fire(ch0 + 2, 0)

            drain(1)
            select(ch0 + 1, 1)

        pltpu.sync_copy(sel_v, out_hbm.at[pl.ds(base, BPW)])

    @functools.partial(
        pl.kernel,
        mesh=mesh,
        compiler_params=pltpu.CompilerParams(use_tc_tiling_on_sc=False),
        out_type=jax.ShapeDtypeStruct((D, B), jnp.float32),
        scratch_types=[
            pltpu.VMEM((BPW,), jnp.int32),
            pltpu.VMEM((8, CH, 128), jnp.int32),
            pltpu.VMEM((D, BPW), jnp.float32),
            pltpu.SemaphoreType.DMA,
        ],
    )
    def kb(idx_hbm, *args):
        tbls = args[:8]
        out_hbm, idx_v, lists_v, rows_v, sem = args[8:]
        wid = lax.axis_index("s") * NC + lax.axis_index("c")
        base = wid * BPW
        pltpu.sync_copy(idx_hbm.at[pl.ds(base, BPW)], idx_v)
        for g in range(BPW // 16):
            v = idx_v[pl.ds(g * 16, 16)]
            vv = jnp.where(v >= R, v - R, lax.bitwise_and(v, _S - 1))
            c = lax.shift_right_logical(vv, _SHIFT)
            w = lax.bitwise_and(vv, _S - 1)
            cw = c * (8 * _S) + w
            for k in range(8):
                lists_v[k, g // 8, pl.ds((g % 8) * 16, 16)] = cw + k * _S

        for r in range(8):
            tbl_r = tbls[r]

            @pl.loop(0, 8)
            def _(k):
                e = 8 * k + r
                copies = [
                    pltpu.async_copy(
                        tbl_r.at[lists_v.at[k, j]],
                        rows_v.at[e, pl.ds(j * 128, 128)],
                        sem,
                    )
                    for j in range(CH)
                ]
                for c2 in copies:
                    c2.wait()

        pltpu.sync_copy(rows_v, out_hbm.at[:, pl.ds(base, BPW)])

    @functools.partial(
        pl.pallas_call,
        grid=(NBLK, 8),
        in_specs=[pl.BlockSpec((8, _S), lambda c, k: (k, R // _S + c))],
        out_specs=tuple(
            pl.BlockSpec((_S,), lambda c, k: (c * 8 + k,)) for _ in range(8)
        ),
        out_shape=tuple(
            jax.ShapeDtypeStruct((8 * V2P,), jnp.float32) for _ in range(8)
        ),
    )
    def depad(in_ref, *out_refs):
        for r in range(8):
            out_refs[r][...] = in_ref[r, :]

    def run(indices, w):
        idx = indices.astype(jnp.int32)
        p1 = w[:R].reshape(R // 8, 8, D)
        p2s = depad(w.T)
        o1 = ka(idx, p1)
        o2t = kb(idx, *p2s)
        return jnp.where((idx < R)[:, None], o1, o2t.T)

    return run


def kernel(indices, embedding_weight):
    V, D = embedding_weight.shape
    (B,) = indices.shape
    return _make_kernels(V, D, B)(indices, embedding_weight)


# speedup vs baseline: 5.5021x; 1.7421x over previous
import functools

import jax
import jax.numpy as jnp
from jax import lax
from jax.experimental import pallas as pl
from jax.experimental.pallas import tpu as pltpu, tpu_sc as plsc

_G = 16
_NBUF = 4


@functools.lru_cache(maxsize=None)
def _make_gather(V, D, B):
    info = plsc.get_sparse_core_info()
    NC, NS = info.num_cores, info.num_subcores
    NW = NC * NS
    BPW = B // NW
    NCH = BPW // _G
    assert B % (NW * _G) == 0 and NCH % _NBUF == 0 and D == 64 and V % 8 == 0
    mesh = plsc.VectorSubcoreMesh(core_axis_name="c", subcore_axis_name="s")

    @functools.partial(
        pl.kernel,
        mesh=mesh,
        out_type=jax.ShapeDtypeStruct((B, D), jnp.float32),
        scratch_types=[
            pltpu.VMEM((BPW,), jnp.int32),
            pltpu.VMEM((_NBUF, _G, 8, D), jnp.float32),
            pltpu.VMEM((_NBUF, _G, D), jnp.float32),
        ] + [pltpu.SemaphoreType.DMA] * 8,
    )
    def gather_kernel(idx_hbm, tbl_hbm, out_hbm, idx_v, tiles_v, osel_v, *sems8):
        wid = lax.axis_index("s") * NC + lax.axis_index("c")
        base = wid * BPW
        pltpu.sync_copy(idx_hbm.at[pl.ds(base, BPW)], idx_v)
        sems, osems = list(sems8[:4]), list(sems8[4:])

        def fire(ch, buf):
            vvec = idx_v[pl.ds(pl.multiple_of(ch * _G, _G), _G)]
            for j in range(_G):
                v8 = lax.shift_right_logical(vvec[j], 3)
                pltpu.async_copy(tbl_hbm.at[v8], tiles_v.at[buf, j], sems[buf])

        def drain(buf):
            pltpu.make_async_copy(
                tbl_hbm.at[pl.ds(0, _G)], tiles_v.at[buf], sems[buf]
            ).wait()

        def owait(buf):
            pltpu.make_async_copy(
                osel_v.at[buf], out_hbm.at[pl.ds(0, _G)], osems[buf]
            ).wait()

        def select_and_out(ch, buf):
            vvec = idx_v[pl.ds(pl.multiple_of(ch * _G, _G), _G)]
            for j in range(_G):
                par = lax.bitwise_and(vvec[j], 7)
                for g in range(D // 16):
                    osel_v[buf, j, pl.ds(g * 16, 16)] = (
                        tiles_v[buf, j, par, pl.ds(g * 16, 16)]
                    )
            pltpu.async_copy(
                osel_v.at[buf], out_hbm.at[pl.ds(base + ch * _G, _G)], osems[buf]
            )

        for b in range(_NBUF - 1):
            fire(b, b)

        @pl.loop(0, NCH // _NBUF)
        def _(t):
            ch0 = t * _NBUF
            for k in range(_NBUF):
                ch = ch0 + k
                drain(k)

                @pl.when(ch >= _NBUF)
                def _():
                    owait(k)

                select_and_out(ch, k)

                @pl.when(ch + _NBUF - 1 < NCH)
                def _():
                    fire(ch + _NBUF - 1, (k + _NBUF - 1) % _NBUF)

        for b in range(_NBUF):
            owait(b)

    return gather_kernel


def kernel(indices, embedding_weight):
    V, D = embedding_weight.shape
    (B,) = indices.shape
    tbl = embedding_weight.reshape(V // 8, 8, D)
    return _make_gather(V, D, B)(indices.astype(jnp.int32), tbl)
